# Initial kernel scaffold; baseline (speedup 1.0000x reference)
import jax
import jax.numpy as jnp
from jax.experimental import pallas as pl


def kernel(x, edge_index, W1, b1, W2, b2):
    raise NotImplementedError("write your pallas kernel here")


# speedup vs baseline: 12.9223x; 12.9223x over previous
import functools

import jax
import jax.numpy as jnp
from jax import lax
from jax.experimental import pallas as pl
from jax.experimental.pallas import tpu as pltpu
from jax.experimental.pallas import tpu_sc as plsc

N = 10000
E = 320000
D = 128
NC = 2
NS = 16
K = 80
EPT = E // (NC * NS)
NCHUNK = EPT // K
NPAD = 10240
RPT = NPAD // NS
ZROWS = 40
HPAD = 10240
HPT = HPAD // NS


@functools.lru_cache(maxsize=1)
def _sc_kernels():
    mesh = plsc.VectorSubcoreMesh(core_axis_name="c", subcore_axis_name="s")

    @functools.partial(
        pl.kernel,
        mesh=mesh,
        out_type=jax.ShapeDtypeStruct((2 * HPAD,), jnp.float32),
        scratch_types=[
            pltpu.VMEM((K,), jnp.int32),
            pltpu.VMEM((K,), jnp.float32),
            pltpu.VMEM((HPT,), jnp.float32),
            pltpu.VMEM_SHARED((HPAD,), jnp.float32),
        ],
    )
    def _sc_hist(edge_ref, out_ref, dst_v, ones_v, zb_v, hist_sh):
        cid = lax.axis_index("c")
        sid = lax.axis_index("s")
        zeros16 = jnp.zeros((16,), jnp.float32)
        ones16 = jnp.ones((16,), jnp.float32)

        def fill_z(i, _):
            zb_v[pl.ds(i * 16, 16)] = zeros16
            return 0

        lax.fori_loop(0, HPT // 16, fill_z, 0)
        for i in range(K // 16):
            ones_v[pl.ds(i * 16, 16)] = ones16
        pltpu.sync_copy(zb_v, hist_sh.at[pl.ds(sid * HPT, HPT)])
        plsc.subcore_barrier()

        ebase = E + cid * (E // NC) + sid * EPT

        def body(j, _):
            pltpu.sync_copy(edge_ref.at[pl.ds(ebase + j * K, K)], dst_v)
            pltpu.sync_copy(ones_v, hist_sh.at[dst_v], add=True)
            return 0

        lax.fori_loop(0, NCHUNK, body, 0)
        plsc.subcore_barrier()
        pltpu.sync_copy(hist_sh.at[pl.ds(sid * HPT, HPT)],
                        out_ref.at[pl.ds(cid * HPAD + sid * HPT, HPT)])

    @functools.partial(
        pl.kernel,
        mesh=mesh,
        out_type=jax.ShapeDtypeStruct((2, NPAD, D), jnp.float32),
        scratch_types=[
            pltpu.VMEM((K,), jnp.int32),
            pltpu.VMEM((K,), jnp.int32),
            pltpu.VMEM((K, D), jnp.float32),
            pltpu.VMEM((ZROWS, D), jnp.float32),
            pltpu.VMEM_SHARED((NPAD, D), jnp.float32),
            pltpu.SemaphoreType.DMA,
        ],
    )
    def _sc_scatter(g_ref, edge_ref, out_ref, src_v, dst_v, rows_v, zb_v,
                    acc_sh, sem):
        cid = lax.axis_index("c")
        sid = lax.axis_index("s")
        zeros16 = jnp.zeros((16,), jnp.float32)

        def fill_z(i, _):
            for l in range(D // 16):
                zb_v[i, pl.ds(l * 16, 16)] = zeros16
            return 0

        lax.fori_loop(0, ZROWS, fill_z, 0)

        def zero_acc(i, _):
            pltpu.sync_copy(zb_v, acc_sh.at[pl.ds(sid * RPT + i * ZROWS, ZROWS)])
            return 0

        lax.fori_loop(0, RPT // ZROWS, zero_acc, 0)
        plsc.subcore_barrier()

        ebase = cid * (E // NC) + sid * EPT

        def body(j, _):
            pltpu.sync_copy(edge_ref.at[pl.ds(ebase + j * K, K)], src_v)
            pltpu.sync_copy(edge_ref.at[pl.ds(E + ebase + j * K, K)], dst_v)
            pltpu.async_copy(g_ref.at[src_v], rows_v, sem).wait()
            pltpu.sync_copy(rows_v, acc_sh.at[dst_v], add=True)
            return 0

        lax.fori_loop(0, NCHUNK, body, 0)
        plsc.subcore_barrier()
        pltpu.sync_copy(acc_sh.at[pl.ds(sid * RPT, RPT)],
                        out_ref.at[cid, pl.ds(sid * RPT, RPT)])

    return _sc_hist, _sc_scatter


RB = 400
NBLK = N // RB


def _lin1_body(x_ref, w_ref, b_ref, h0_ref, h1_ref, g_ref, dinv_ref):
    deg = h0_ref[...] + h1_ref[...] + 1.0
    dinv = lax.rsqrt(deg)
    h = jnp.dot(x_ref[...], w_ref[...], preferred_element_type=jnp.float32)
    g_ref[...] = (h + b_ref[...]) * dinv
    dinv_ref[...] = dinv


def _mid_body(a0_ref, a1_ref, g1_ref, dinv_ref, w_ref, b_ref, g2_ref):
    dinv = dinv_ref[...]
    x2 = jnp.maximum((a0_ref[0] + a1_ref[0] + g1_ref[...]) * dinv, 0.0)
    h = jnp.dot(x2, w_ref[...], preferred_element_type=jnp.float32)
    g2_ref[...] = (h + b_ref[...]) * dinv


def _fin_body(a0_ref, a1_ref, g2_ref, dinv_ref, out_ref):
    out_ref[...] = (a0_ref[0] + a1_ref[0] + g2_ref[...]) * dinv_ref[...]


_row_spec = pl.BlockSpec((RB, D), lambda i: (i, 0))
_acc0_spec = pl.BlockSpec((1, RB, D), lambda i: (0, i, 0))
_acc1_spec = pl.BlockSpec((1, RB, D), lambda i: (1, i, 0))
_col_spec = pl.BlockSpec((RB, 1), lambda i: (i, 0))
_w_spec = pl.BlockSpec((D, D), lambda i: (0, 0))
_b_spec = pl.BlockSpec((1, D), lambda i: (0, 0))

_lin1 = pl.pallas_call(
    _lin1_body,
    grid=(NBLK,),
    in_specs=[_row_spec, _w_spec, _b_spec, _col_spec, _col_spec],
    out_specs=[_row_spec, _col_spec],
    out_shape=[
        jax.ShapeDtypeStruct((N, D), jnp.float32),
        jax.ShapeDtypeStruct((N, 1), jnp.float32),
    ],
)

_mid = pl.pallas_call(
    _mid_body,
    grid=(NBLK,),
    in_specs=[_acc0_spec, _acc1_spec, _row_spec, _col_spec, _w_spec, _b_spec],
    out_specs=_row_spec,
    out_shape=jax.ShapeDtypeStruct((N, D), jnp.float32),
)

_fin = pl.pallas_call(
    _fin_body,
    grid=(NBLK,),
    in_specs=[_acc0_spec, _acc1_spec, _row_spec, _col_spec],
    out_specs=_row_spec,
    out_shape=jax.ShapeDtypeStruct((N, D), jnp.float32),
)


def kernel(x, edge_index, W1, b1, W2, b2):
    _sc_hist, _sc_scatter = _sc_kernels()
    ef = edge_index.reshape(-1)
    hist = _sc_hist(ef)
    h0 = hist[0:N].reshape(N, 1)
    h1 = hist[HPAD:HPAD + N].reshape(N, 1)
    b1r = b1.reshape(1, D)
    b2r = b2.reshape(1, D)
    g1, dinv = _lin1(x, W1, b1r, h0, h1)
    acc1 = _sc_scatter(g1, ef)
    g2 = _mid(acc1, acc1, g1, dinv, W2, b2r)
    acc2 = _sc_scatter(g2, ef)
    out = _fin(acc2, acc2, g2, dinv)
    return out


# speedup vs baseline: 28.7844x; 2.2275x over previous
import functools

import jax
import jax.numpy as jnp
from jax import lax
from jax.experimental import pallas as pl
from jax.experimental.pallas import tpu as pltpu
from jax.experimental.pallas import tpu_sc as plsc

N = 10000
E = 320000
D = 128
NC = 2
NS = 16
K = 40
EPT = E // (NC * NS)
NCHUNK = EPT // K
NPAD = 10240
RPT = NPAD // NS
HPAD = 10240
HPT = HPAD // NS


HBUF = 5
HGRP = NCHUNK // HBUF
SBUF = 4
SGRP = (NCHUNK - 2) // 8


@functools.lru_cache(maxsize=1)
def _sc_kernels():
    mesh = plsc.VectorSubcoreMesh(core_axis_name="c", subcore_axis_name="s")

    @functools.partial(
        pl.kernel,
        mesh=mesh,
        out_type=jax.ShapeDtypeStruct((2 * HPAD,), jnp.float32),
        scratch_types=[
            pltpu.VMEM((NCHUNK, K), jnp.int32),
            pltpu.VMEM((K,), jnp.float32),
            pltpu.VMEM((HPT,), jnp.float32),
            pltpu.VMEM_SHARED((HPAD,), jnp.float32),
            pltpu.SemaphoreType.DMA,
        ],
    )
    def _sc_hist(edge_ref, out_ref, dsts_v, ones_v, zb_v, hist_sh, hsem):
        cid = lax.axis_index("c")
        sid = lax.axis_index("s")
        w = cid * NS + sid
        zeros16 = jnp.zeros((16,), jnp.float32)
        ones16 = jnp.ones((16,), jnp.float32)

        pltpu.sync_copy(edge_ref.at[1, w], dsts_v)

        def fill_z(i, _):
            zb_v[pl.ds(i * 16, 16)] = zeros16
            return 0

        lax.fori_loop(0, HPT // 16, fill_z, 0)
        for off in (0, 16, K - 16):
            ones_v[pl.ds(off, 16)] = ones16
        pltpu.sync_copy(zb_v, hist_sh.at[pl.ds(sid * HPT, HPT)])
        plsc.subcore_barrier()

        def body(g, _):
            for b in range(HBUF):
                pltpu.async_copy(ones_v, hist_sh.at[dsts_v.at[g * HBUF + b]],
                                 hsem, add=True)

            @pl.when(g > 0)
            def _drain():
                for _ in range(HBUF):
                    pltpu.make_async_copy(
                        ones_v, hist_sh.at[pl.ds(0, K)], hsem).wait()

            return 0

        lax.fori_loop(0, HGRP, body, 0)
        for _ in range(HBUF):
            pltpu.make_async_copy(ones_v, hist_sh.at[pl.ds(0, K)], hsem).wait()
        plsc.subcore_barrier()
        pltpu.sync_copy(hist_sh.at[pl.ds(sid * HPT, HPT)],
                        out_ref.at[pl.ds(cid * HPAD + sid * HPT, HPT)])

    @functools.partial(
        pl.kernel,
        mesh=mesh,
        out_type=jax.ShapeDtypeStruct((2, NPAD, D), jnp.float32),
        scratch_types=[
            pltpu.VMEM((2, 8, K), jnp.int32),
            pltpu.VMEM((2, 8, K), jnp.int32),
            pltpu.VMEM((SBUF, K, D), jnp.float32),
            pltpu.VMEM_SHARED((NPAD, D), jnp.float32),
        ] + [pltpu.SemaphoreType.DMA] * (2 * SBUF + 1),
    )
    def _sc_scatter(g_ref, em_ref, et_ref, out_ref, srcb, dstb, rows_v,
                    acc_sh, *sems):
        gsem = sems[:SBUF]
        ssem = sems[SBUF:2 * SBUF]
        isem = sems[2 * SBUF]
        cid = lax.axis_index("c")
        sid = lax.axis_index("s")
        w = cid * NS + sid
        zeros16 = jnp.zeros((16,), jnp.float32)

        def idx_load(sg, slot):
            pltpu.async_copy(em_ref.at[0, w, sg], srcb.at[slot], isem)
            pltpu.async_copy(em_ref.at[1, w, sg], dstb.at[slot], isem)

        def idx_wait(slot):
            pltpu.make_async_copy(em_ref.at[0, w, 0], srcb.at[slot], isem).wait()
            pltpu.make_async_copy(em_ref.at[1, w, 0], dstb.at[slot], isem).wait()

        def gwait(b):
            pltpu.make_async_copy(g_ref.at[pl.ds(0, K)], rows_v.at[b],
                                  gsem[b]).wait()

        def swait(b):
            pltpu.make_async_copy(rows_v.at[b], acc_sh.at[pl.ds(0, K)],
                                  ssem[b]).wait()

        idx_load(0, 0)

        def fill_z(i, _):
            for l in range(D // 16):
                rows_v[0, i, pl.ds(l * 16, 16)] = zeros16
            return 0

        lax.fori_loop(0, K, fill_z, 0)

        def zero_acc(i, _):
            pltpu.sync_copy(rows_v.at[0], acc_sh.at[pl.ds(sid * RPT + i * K, K)])
            return 0

        lax.fori_loop(0, RPT // K, zero_acc, 0)

        idx_wait(0)
        for b in range(SBUF):
            pltpu.async_copy(g_ref.at[srcb.at[0, b]], rows_v.at[b], gsem[b])
        idx_load(1, 1)
        plsc.subcore_barrier()

        def body(sg, _):
            slot = lax.rem(sg, 2)
            for b in range(SBUF):
                gwait(b)
                pltpu.async_copy(rows_v.at[b], acc_sh.at[dstb.at[slot, b]],
                                 ssem[b], add=True)
            for b in range(SBUF):
                swait(b)
                pltpu.async_copy(g_ref.at[srcb.at[slot, SBUF + b]],
                                 rows_v.at[b], gsem[b])
            for b in range(SBUF):
                gwait(b)
                pltpu.async_copy(rows_v.at[b],
                                 acc_sh.at[dstb.at[slot, SBUF + b]],
                                 ssem[b], add=True)

            @pl.when(sg < SGRP - 1)
            def _refill():
                nslot = 1 - slot
                idx_wait(nslot)
                for b in range(SBUF):
                    swait(b)
                    pltpu.async_copy(g_ref.at[srcb.at[nslot, b]],
                                     rows_v.at[b], gsem[b])

                @pl.when(sg < SGRP - 2)
                def _next_idx():
                    idx_load(sg + 2, slot)

            return 0

        lax.fori_loop(0, SGRP, body, 0)
        for b in range(SBUF):
            swait(b)

        pltpu.sync_copy(et_ref.at[0, w], srcb.at[0, pl.ds(0, 2)])
        pltpu.sync_copy(et_ref.at[1, w], dstb.at[0, pl.ds(0, 2)])
        for t in range(2):
            pltpu.async_copy(g_ref.at[srcb.at[0, t]], rows_v.at[t],
                             gsem[t]).wait()
            pltpu.sync_copy(rows_v.at[t], acc_sh.at[dstb.at[0, t]], add=True)

        plsc.subcore_barrier()
        pltpu.sync_copy(acc_sh.at[pl.ds(sid * RPT, RPT)],
                        out_ref.at[cid, pl.ds(sid * RPT, RPT)])

    return _sc_hist, _sc_scatter


RB = 400
NBLK = N // RB


def _lin1_body(x_ref, w_ref, b_ref, h0_ref, h1_ref, g_ref, dinv_ref):
    deg = h0_ref[...] + h1_ref[...] + 1.0
    dinv = lax.rsqrt(deg)
    h = jnp.dot(x_ref[...], w_ref[...], preferred_element_type=jnp.float32)
    g_ref[...] = (h + b_ref[...]) * dinv
    dinv_ref[...] = dinv


def _mid_body(a0_ref, a1_ref, g1_ref, dinv_ref, w_ref, b_ref, g2_ref):
    dinv = dinv_ref[...]
    x2 = jnp.maximum((a0_ref[0] + a1_ref[0] + g1_ref[...]) * dinv, 0.0)
    h = jnp.dot(x2, w_ref[...], preferred_element_type=jnp.float32)
    g2_ref[...] = (h + b_ref[...]) * dinv


def _fin_body(a0_ref, a1_ref, g2_ref, dinv_ref, out_ref):
    out_ref[...] = (a0_ref[0] + a1_ref[0] + g2_ref[...]) * dinv_ref[...]


_row_spec = pl.BlockSpec((RB, D), lambda i: (i, 0))
_acc0_spec = pl.BlockSpec((1, RB, D), lambda i: (0, i, 0))
_acc1_spec = pl.BlockSpec((1, RB, D), lambda i: (1, i, 0))
_col_spec = pl.BlockSpec((RB, 1), lambda i: (i, 0))
_w_spec = pl.BlockSpec((D, D), lambda i: (0, 0))
_b_spec = pl.BlockSpec((1, D), lambda i: (0, 0))

_lin1 = pl.pallas_call(
    _lin1_body,
    grid=(NBLK,),
    in_specs=[_row_spec, _w_spec, _b_spec, _col_spec, _col_spec],
    out_specs=[_row_spec, _col_spec],
    out_shape=[
        jax.ShapeDtypeStruct((N, D), jnp.float32),
        jax.ShapeDtypeStruct((N, 1), jnp.float32),
    ],
)

_mid = pl.pallas_call(
    _mid_body,
    grid=(NBLK,),
    in_specs=[_acc0_spec, _acc1_spec, _row_spec, _col_spec, _w_spec, _b_spec],
    out_specs=_row_spec,
    out_shape=jax.ShapeDtypeStruct((N, D), jnp.float32),
)

_fin = pl.pallas_call(
    _fin_body,
    grid=(NBLK,),
    in_specs=[_acc0_spec, _acc1_spec, _row_spec, _col_spec],
    out_specs=_row_spec,
    out_shape=jax.ShapeDtypeStruct((N, D), jnp.float32),
)


def kernel(x, edge_index, W1, b1, W2, b2):
    _sc_hist, _sc_scatter = _sc_kernels()
    ei = edge_index.reshape(2, NC * NS, NCHUNK, K)
    em = ei[:, :, :NCHUNK - 2].reshape(2, NC * NS, SGRP, 8, K)
    et = ei[:, :, NCHUNK - 2:]
    hist = _sc_hist(ei)
    h0 = hist[0:N].reshape(N, 1)
    h1 = hist[HPAD:HPAD + N].reshape(N, 1)
    b1r = b1.reshape(1, D)
    b2r = b2.reshape(1, D)
    g1, dinv = _lin1(x, W1, b1r, h0, h1)
    acc1 = _sc_scatter(g1, em, et)
    g2 = _mid(acc1, acc1, g1, dinv, W2, b2r)
    acc2 = _sc_scatter(g2, em, et)
    out = _fin(acc2, acc2, g2, dinv)
    return out


# speedup vs baseline: 30.0777x; 1.0449x over previous
import functools

import jax
import jax.numpy as jnp
from jax import lax
from jax.experimental import pallas as pl
from jax.experimental.pallas import tpu as pltpu
from jax.experimental.pallas import tpu_sc as plsc

N = 10000
E = 320000
D = 128
NC = 2
NS = 16
K = 40
EPT = E // (NC * NS)
NCHUNK = EPT // K
NPAD = 10240
RPT = NPAD // NS
HPAD = 10240
HPT = HPAD // NS


HBUF = 5
HGRP = NCHUNK // HBUF
SBUF = 8
SGRP = (NCHUNK - 2) // 8


@functools.lru_cache(maxsize=1)
def _sc_kernels():
    mesh = plsc.VectorSubcoreMesh(core_axis_name="c", subcore_axis_name="s")

    @functools.partial(
        pl.kernel,
        mesh=mesh,
        out_type=jax.ShapeDtypeStruct((2 * HPAD,), jnp.float32),
        scratch_types=[
            pltpu.VMEM((NCHUNK, K), jnp.int32),
            pltpu.VMEM((K,), jnp.float32),
            pltpu.VMEM((HPT,), jnp.float32),
            pltpu.VMEM_SHARED((HPAD,), jnp.float32),
            pltpu.SemaphoreType.DMA,
        ],
    )
    def _sc_hist(edge_ref, out_ref, dsts_v, ones_v, zb_v, hist_sh, hsem):
        cid = lax.axis_index("c")
        sid = lax.axis_index("s")
        w = cid * NS + sid
        zeros16 = jnp.zeros((16,), jnp.float32)
        ones16 = jnp.ones((16,), jnp.float32)

        pltpu.sync_copy(edge_ref.at[1, w], dsts_v)

        def fill_z(i, _):
            zb_v[pl.ds(i * 16, 16)] = zeros16
            return 0

        lax.fori_loop(0, HPT // 16, fill_z, 0)
        for off in (0, 16, K - 16):
            ones_v[pl.ds(off, 16)] = ones16
        pltpu.sync_copy(zb_v, hist_sh.at[pl.ds(sid * HPT, HPT)])
        plsc.subcore_barrier()

        def body(g, _):
            for b in range(HBUF):
                pltpu.async_copy(ones_v, hist_sh.at[dsts_v.at[g * HBUF + b]],
                                 hsem, add=True)

            @pl.when(g > 0)
            def _drain():
                for _ in range(HBUF):
                    pltpu.make_async_copy(
                        ones_v, hist_sh.at[pl.ds(0, K)], hsem).wait()

            return 0

        lax.fori_loop(0, HGRP, body, 0)
        for _ in range(HBUF):
            pltpu.make_async_copy(ones_v, hist_sh.at[pl.ds(0, K)], hsem).wait()
        plsc.subcore_barrier()
        pltpu.sync_copy(hist_sh.at[pl.ds(sid * HPT, HPT)],
                        out_ref.at[pl.ds(cid * HPAD + sid * HPT, HPT)])

    @functools.partial(
        pl.kernel,
        mesh=mesh,
        out_type=jax.ShapeDtypeStruct((2, NPAD, D), jnp.float32),
        scratch_types=[
            pltpu.VMEM((2, 8, K), jnp.int32),
            pltpu.VMEM((2, 8, K), jnp.int32),
            pltpu.VMEM((SBUF, K, D), jnp.float32),
            pltpu.VMEM_SHARED((NPAD, D), jnp.float32),
        ] + [pltpu.SemaphoreType.DMA] * (2 * SBUF + 1),
    )
    def _sc_scatter(g_ref, em_ref, et_ref, out_ref, srcb, dstb, rows_v,
                    acc_sh, *sems):
        gsem = sems[:SBUF]
        ssem = sems[SBUF:2 * SBUF]
        isem = sems[2 * SBUF]
        cid = lax.axis_index("c")
        sid = lax.axis_index("s")
        w = cid * NS + sid
        zeros16 = jnp.zeros((16,), jnp.float32)

        def idx_load(sg, slot):
            pltpu.async_copy(em_ref.at[0, w, sg], srcb.at[slot], isem)
            pltpu.async_copy(em_ref.at[1, w, sg], dstb.at[slot], isem)

        def idx_wait(slot):
            pltpu.make_async_copy(em_ref.at[0, w, 0], srcb.at[slot], isem).wait()
            pltpu.make_async_copy(em_ref.at[1, w, 0], dstb.at[slot], isem).wait()

        def gwait(b):
            pltpu.make_async_copy(g_ref.at[pl.ds(0, K)], rows_v.at[b],
                                  gsem[b]).wait()

        def swait(b):
            pltpu.make_async_copy(rows_v.at[b], acc_sh.at[pl.ds(0, K)],
                                  ssem[b]).wait()

        idx_load(0, 0)

        def fill_z(i, _):
            for l in range(D // 16):
                rows_v[0, i, pl.ds(l * 16, 16)] = zeros16
            return 0

        lax.fori_loop(0, K, fill_z, 0)

        def zero_acc(i, _):
            pltpu.sync_copy(rows_v.at[0], acc_sh.at[pl.ds(sid * RPT + i * K, K)])
            return 0

        lax.fori_loop(0, RPT // K, zero_acc, 0)

        idx_wait(0)
        for b in range(SBUF):
            pltpu.async_copy(g_ref.at[srcb.at[0, b]], rows_v.at[b], gsem[b])
        idx_load(1, 1)
        plsc.subcore_barrier()

        def body(sg, _):
            slot = lax.rem(sg, 2)
            for b in range(SBUF):
                gwait(b)
                pltpu.async_copy(rows_v.at[b], acc_sh.at[dstb.at[slot, b]],
                                 ssem[b], add=True)

            @pl.when(sg < SGRP - 1)
            def _refill():
                nslot = 1 - slot
                idx_wait(nslot)
                for b in range(SBUF):
                    swait(b)
                    pltpu.async_copy(g_ref.at[srcb.at[nslot, b]],
                                     rows_v.at[b], gsem[b])

                @pl.when(sg < SGRP - 2)
                def _next_idx():
                    idx_load(sg + 2, slot)

            return 0

        lax.fori_loop(0, SGRP, body, 0)
        for b in range(SBUF):
            swait(b)

        pltpu.sync_copy(et_ref.at[0, w], srcb.at[0, pl.ds(0, 2)])
        pltpu.sync_copy(et_ref.at[1, w], dstb.at[0, pl.ds(0, 2)])
        for t in range(2):
            pltpu.async_copy(g_ref.at[srcb.at[0, t]], rows_v.at[t],
                             gsem[t]).wait()
            pltpu.sync_copy(rows_v.at[t], acc_sh.at[dstb.at[0, t]], add=True)

        plsc.subcore_barrier()
        pltpu.sync_copy(acc_sh.at[pl.ds(sid * RPT, RPT)],
                        out_ref.at[cid, pl.ds(sid * RPT, RPT)])

    return _sc_hist, _sc_scatter


RB = 400
NBLK = N // RB


def _lin1_body(x_ref, w_ref, b_ref, h0_ref, h1_ref, g_ref, dinv_ref):
    deg = h0_ref[...] + h1_ref[...] + 1.0
    dinv = lax.rsqrt(deg)
    h = jnp.dot(x_ref[...], w_ref[...], preferred_element_type=jnp.float32)
    g_ref[...] = (h + b_ref[...]) * dinv
    dinv_ref[...] = dinv


def _mid_body(a0_ref, a1_ref, g1_ref, dinv_ref, w_ref, b_ref, g2_ref):
    dinv = dinv_ref[...]
    x2 = jnp.maximum((a0_ref[0] + a1_ref[0] + g1_ref[...]) * dinv, 0.0)
    h = jnp.dot(x2, w_ref[...], preferred_element_type=jnp.float32)
    g2_ref[...] = (h + b_ref[...]) * dinv


def _fin_body(a0_ref, a1_ref, g2_ref, dinv_ref, out_ref):
    out_ref[...] = (a0_ref[0] + a1_ref[0] + g2_ref[...]) * dinv_ref[...]


_row_spec = pl.BlockSpec((RB, D), lambda i: (i, 0))
_acc0_spec = pl.BlockSpec((1, RB, D), lambda i: (0, i, 0))
_acc1_spec = pl.BlockSpec((1, RB, D), lambda i: (1, i, 0))
_col_spec = pl.BlockSpec((RB, 1), lambda i: (i, 0))
_w_spec = pl.BlockSpec((D, D), lambda i: (0, 0))
_b_spec = pl.BlockSpec((1, D), lambda i: (0, 0))

_lin1 = pl.pallas_call(
    _lin1_body,
    grid=(NBLK,),
    in_specs=[_row_spec, _w_spec, _b_spec, _col_spec, _col_spec],
    out_specs=[_row_spec, _col_spec],
    out_shape=[
        jax.ShapeDtypeStruct((N, D), jnp.float32),
        jax.ShapeDtypeStruct((N, 1), jnp.float32),
    ],
)

_mid = pl.pallas_call(
    _mid_body,
    grid=(NBLK,),
    in_specs=[_acc0_spec, _acc1_spec, _row_spec, _col_spec, _w_spec, _b_spec],
    out_specs=_row_spec,
    out_shape=jax.ShapeDtypeStruct((N, D), jnp.float32),
)

_fin = pl.pallas_call(
    _fin_body,
    grid=(NBLK,),
    in_specs=[_acc0_spec, _acc1_spec, _row_spec, _col_spec],
    out_specs=_row_spec,
    out_shape=jax.ShapeDtypeStruct((N, D), jnp.float32),
)


def kernel(x, edge_index, W1, b1, W2, b2):
    _sc_hist, _sc_scatter = _sc_kernels()
    ei = edge_index.reshape(2, NC * NS, NCHUNK, K)
    em = ei[:, :, :NCHUNK - 2].reshape(2, NC * NS, SGRP, 8, K)
    et = ei[:, :, NCHUNK - 2:]
    hist = _sc_hist(ei)
    h0 = hist[0:N].reshape(N, 1)
    h1 = hist[HPAD:HPAD + N].reshape(N, 1)
    b1r = b1.reshape(1, D)
    b2r = b2.reshape(1, D)
    g1, dinv = _lin1(x, W1, b1r, h0, h1)
    acc1 = _sc_scatter(g1, em, et)
    g2 = _mid(acc1, acc1, g1, dinv, W2, b2r)
    acc2 = _sc_scatter(g2, em, et)
    out = _fin(acc2, acc2, g2, dinv)
    return out
